# baseline (speedup 1.0000x reference)
import functools

import jax
import jax.numpy as jnp
from jax import lax
from jax.experimental import pallas as pl
from jax.experimental.pallas import tpu as pltpu
from jax.experimental.pallas import tpu_sc as plsc

_N = 100000
_D = 128
_V = 257
_K = 128
_FULL = _N // _K
_TAIL = _N - _FULL * _K
_NW = 32
_IDXN = 3200
_L = 16


def _compute(zin_v, zout_v, ii, io, acc_c, ib, ngroups):
    riota = lax.iota(jnp.int32, _L)

    def group(g, carry):
        din = ii[pl.ds(ib + g * _L, _L)]
        dout = io[pl.ds(ib + g * _L, _L)]
        i_in0 = din * _D
        i_out0 = dout * _D
        s_0 = (g * _L + riota) * _D

        def col(cc, c2):
            ii_v, io_v, s_v = c2
            v = plsc.load_gather(zin_v, [ii_v]) + plsc.load_gather(zout_v, [io_v])
            plsc.addupdate_scatter(acc_c, [s_v], v)
            return (ii_v + 1, io_v + 1, s_v + 1)

        lax.fori_loop(0, _D, col, (i_in0, i_out0, s_0), unroll=8)
        return carry

    lax.fori_loop(0, ngroups, group, 0, unroll=False)


def _sc_body(x_hbm, din_hbm, dout_hbm, zin_hbm, zout_hbm, out_hbm,
             zin_v, zout_v, ii, io, acc0, acc1, ls0, ls1, ss0, ss1):
    wid = lax.axis_index("s") * 2 + lax.axis_index("c")
    start_chunk = 24 * wid + jnp.minimum(wid, 13)
    count = jnp.where(wid < 13, 25, 24)
    base_row = start_chunk * _K
    idx_s0 = jnp.minimum(base_row, _N - _IDXN)
    off = base_row - idx_s0

    c1 = pltpu.async_copy(zin_hbm, zin_v, ls0)
    c2 = pltpu.async_copy(zout_hbm, zout_v, ls0)
    c3 = pltpu.async_copy(din_hbm.at[pl.ds(idx_s0, _IDXN)], ii, ls1)
    c4 = pltpu.async_copy(dout_hbm.at[pl.ds(idx_s0, _IDXN)], io, ls1)
    c1.wait(); c2.wait(); c3.wait(); c4.wait()

    accs = (acc0, acc1)
    lsems = (ls0, ls1)
    ssems = (ss0, ss1)

    def flat_chunk(t):
        return pl.ds((base_row + t * _K) * _D, _K * _D)

    pltpu.async_copy(x_hbm.at[flat_chunk(0)], acc0, ls0)

    def pair_body(tp, carry):
        for b in (0, 1):
            t = tp * 2 + b
            nb = 1 - b

            @pl.when(t + 1 < count)
            def _():
                @pl.when(t >= 1)
                def _():
                    pltpu.make_async_copy(
                        accs[nb], out_hbm.at[flat_chunk(0)], ssems[nb]).wait()

                pltpu.async_copy(x_hbm.at[flat_chunk(t + 1)], accs[nb],
                                 lsems[nb])

            @pl.when(t < count)
            def _():
                pltpu.make_async_copy(
                    x_hbm.at[flat_chunk(0)], accs[b], lsems[b]).wait()
                _compute(zin_v, zout_v, ii, io, accs[b], off + t * _K,
                         _K // _L)
                pltpu.async_copy(accs[b], out_hbm.at[flat_chunk(t)], ssems[b])

        return carry

    lax.fori_loop(0, 13, pair_body, 0, unroll=False)

    pltpu.make_async_copy(acc0, out_hbm.at[flat_chunk(0)], ss0).wait()
    pltpu.make_async_copy(acc1, out_hbm.at[flat_chunk(0)], ss1).wait()

    @pl.when(wid == _NW - 1)
    def _():
        tail = pl.ds(_FULL * _K * _D, _TAIL * _D)
        pltpu.sync_copy(x_hbm.at[tail], acc0.at[pl.ds(0, _TAIL * _D)])
        _compute(zin_v, zout_v, ii, io, acc0, off + 24 * _K, _TAIL // _L)
        pltpu.sync_copy(acc0.at[pl.ds(0, _TAIL * _D)], out_hbm.at[tail])


@jax.jit
def _centrality(x2, din, dout, z_in, z_out):
    mesh = plsc.VectorSubcoreMesh(core_axis_name="c", subcore_axis_name="s")
    fn = functools.partial(
        pl.kernel,
        mesh=mesh,
        compiler_params=pltpu.CompilerParams(needs_layout_passes=False),
        out_type=jax.ShapeDtypeStruct((_N * _D,), jnp.float32),
        scratch_types=[
            pltpu.VMEM((_V * _D,), jnp.float32),
            pltpu.VMEM((_V * _D,), jnp.float32),
            pltpu.VMEM((_IDXN,), jnp.int32),
            pltpu.VMEM((_IDXN,), jnp.int32),
            pltpu.VMEM((_K * _D,), jnp.float32),
            pltpu.VMEM((_K * _D,), jnp.float32),
            pltpu.SemaphoreType.DMA,
            pltpu.SemaphoreType.DMA,
            pltpu.SemaphoreType.DMA,
            pltpu.SemaphoreType.DMA,
        ],
    )(_sc_body)
    return fn(x2, din, dout, z_in, z_out)


def kernel(x, in_degree, out_degree, z_in, z_out):
    x2 = x.reshape(_N * _D)
    out2 = _centrality(x2, in_degree.astype(jnp.int32),
                       out_degree.astype(jnp.int32), z_in.reshape(_V * _D),
                       z_out.reshape(_V * _D))
    return out2.reshape(x.shape)


# speedup vs baseline: 1.2778x; 1.2778x over previous
import functools

import jax
import jax.numpy as jnp
from jax import lax
from jax.experimental import pallas as pl
from jax.experimental.pallas import tpu as pltpu
from jax.experimental.pallas import tpu_sc as plsc

_N = 100000
_D = 128
_V = 257
_K = 128
_FULL = _N // _K
_TAIL = _N - _FULL * _K
_NW = 32
_IDXN = 3200
_L = 16


def _compute(zin_v, zout_v, ii, io, acc_c, ib, ngroups):
    riota = lax.iota(jnp.int32, _L)

    def group(g, carry):
        din = ii[pl.ds(ib + g * _L, _L)]
        dout = io[pl.ds(ib + g * _L, _L)]
        i_in0 = din * _D
        i_out0 = dout * _D
        s_0 = (g * _L + riota) * _D

        @plsc.parallel_loop(0, _D, carry=(i_in0, i_out0, s_0), unroll=8)
        def col(cc, c2):
            ii_v, io_v, s_v = c2
            v = plsc.load_gather(zin_v, [ii_v]) + plsc.load_gather(zout_v, [io_v])
            plsc.addupdate_scatter(acc_c, [s_v], v)
            return (ii_v + 1, io_v + 1, s_v + 1)
        return carry

    lax.fori_loop(0, ngroups, group, 0, unroll=False)


def _sc_body(x_hbm, din_hbm, dout_hbm, zin_hbm, zout_hbm, out_hbm,
             zin_v, zout_v, ii, io, acc0, acc1, ls0, ls1, ss0, ss1):
    wid = lax.axis_index("s") * 2 + lax.axis_index("c")
    start_chunk = 24 * wid + jnp.minimum(wid, 13)
    count = jnp.where(wid < 13, 25, 24)
    base_row = start_chunk * _K
    idx_s0 = jnp.minimum(base_row, _N - _IDXN)
    off = base_row - idx_s0

    c1 = pltpu.async_copy(zin_hbm, zin_v, ls0)
    c2 = pltpu.async_copy(zout_hbm, zout_v, ls0)
    c3 = pltpu.async_copy(din_hbm.at[pl.ds(idx_s0, _IDXN)], ii, ls1)
    c4 = pltpu.async_copy(dout_hbm.at[pl.ds(idx_s0, _IDXN)], io, ls1)
    c1.wait(); c2.wait(); c3.wait(); c4.wait()

    accs = (acc0, acc1)
    lsems = (ls0, ls1)
    ssems = (ss0, ss1)

    def flat_chunk(t):
        return pl.ds((base_row + t * _K) * _D, _K * _D)

    pltpu.async_copy(x_hbm.at[flat_chunk(0)], acc0, ls0)

    def pair_body(tp, carry):
        for b in (0, 1):
            t = tp * 2 + b
            nb = 1 - b

            @pl.when(t + 1 < count)
            def _():
                @pl.when(t >= 1)
                def _():
                    pltpu.make_async_copy(
                        accs[nb], out_hbm.at[flat_chunk(0)], ssems[nb]).wait()

                pltpu.async_copy(x_hbm.at[flat_chunk(t + 1)], accs[nb],
                                 lsems[nb])

            @pl.when(t < count)
            def _():
                pltpu.make_async_copy(
                    x_hbm.at[flat_chunk(0)], accs[b], lsems[b]).wait()
                _compute(zin_v, zout_v, ii, io, accs[b], off + t * _K,
                         _K // _L)
                pltpu.async_copy(accs[b], out_hbm.at[flat_chunk(t)], ssems[b])

        return carry

    lax.fori_loop(0, 13, pair_body, 0, unroll=False)

    pltpu.make_async_copy(acc0, out_hbm.at[flat_chunk(0)], ss0).wait()
    pltpu.make_async_copy(acc1, out_hbm.at[flat_chunk(0)], ss1).wait()

    @pl.when(wid == _NW - 1)
    def _():
        tail = pl.ds(_FULL * _K * _D, _TAIL * _D)
        pltpu.sync_copy(x_hbm.at[tail], acc0.at[pl.ds(0, _TAIL * _D)])
        _compute(zin_v, zout_v, ii, io, acc0, off + 24 * _K, _TAIL // _L)
        pltpu.sync_copy(acc0.at[pl.ds(0, _TAIL * _D)], out_hbm.at[tail])


@jax.jit
def _centrality(x2, din, dout, z_in, z_out):
    mesh = plsc.VectorSubcoreMesh(core_axis_name="c", subcore_axis_name="s")
    fn = functools.partial(
        pl.kernel,
        mesh=mesh,
        compiler_params=pltpu.CompilerParams(needs_layout_passes=False),
        out_type=jax.ShapeDtypeStruct((_N * _D,), jnp.float32),
        scratch_types=[
            pltpu.VMEM((_V * _D,), jnp.float32),
            pltpu.VMEM((_V * _D,), jnp.float32),
            pltpu.VMEM((_IDXN,), jnp.int32),
            pltpu.VMEM((_IDXN,), jnp.int32),
            pltpu.VMEM((_K * _D,), jnp.float32),
            pltpu.VMEM((_K * _D,), jnp.float32),
            pltpu.SemaphoreType.DMA,
            pltpu.SemaphoreType.DMA,
            pltpu.SemaphoreType.DMA,
            pltpu.SemaphoreType.DMA,
        ],
    )(_sc_body)
    return fn(x2, din, dout, z_in, z_out)


def kernel(x, in_degree, out_degree, z_in, z_out):
    x2 = x.reshape(_N * _D)
    out2 = _centrality(x2, in_degree.astype(jnp.int32),
                       out_degree.astype(jnp.int32), z_in.reshape(_V * _D),
                       z_out.reshape(_V * _D))
    return out2.reshape(x.shape)


# speedup vs baseline: 8.7879x; 6.8771x over previous
import functools

import jax
import jax.numpy as jnp
from jax import lax
from jax.experimental import pallas as pl
from jax.experimental.pallas import tpu as pltpu
from jax.experimental.pallas import tpu_sc as plsc

_N = 100000
_D = 128
_V = 257
_K = 128
_FULL = _N // _K
_TAIL = _N - _FULL * _K
_NW = 32
_IDXN = 3200
_L = 16


def _compute(zin_v, zout_v, ii, io, acc_c, ib, ngroups):
    iotas = [lax.iota(jnp.int32, _L) + k * _L for k in range(_D // _L)]

    def group(g, carry):
        din = ii[pl.ds(ib + g * _L, _L)] * _D
        dout = io[pl.ds(ib + g * _L, _L)] * _D
        gbase = g * _L * _D

        @plsc.parallel_loop(0, _L, unroll=2)
        def node(j):
            jsp = lax.broadcast(j, (_L,))
            bi = jnp.take_along_axis(din, jsp, axis=0)
            bo = jnp.take_along_axis(dout, jsp, axis=0)
            for k in range(_D // _L):
                v = (plsc.load_gather(zin_v, [bi + iotas[k]])
                     + plsc.load_gather(zout_v, [bo + iotas[k]]))
                plsc.addupdate(acc_c.at[pl.ds(gbase + j * _D + k * _L, _L)], v)

        return carry

    lax.fori_loop(0, ngroups, group, 0, unroll=False)


def _sc_body(x_hbm, din_hbm, dout_hbm, zin_hbm, zout_hbm, out_hbm,
             zin_v, zout_v, ii, io, acc0, acc1, ls0, ls1, ss0, ss1):
    wid = lax.axis_index("s") * 2 + lax.axis_index("c")
    start_chunk = 24 * wid + jnp.minimum(wid, 13)
    count = jnp.where(wid < 13, 25, 24)
    base_row = start_chunk * _K
    idx_s0 = jnp.minimum(base_row, _N - _IDXN)
    off = base_row - idx_s0

    c1 = pltpu.async_copy(zin_hbm, zin_v, ls0)
    c2 = pltpu.async_copy(zout_hbm, zout_v, ls0)
    c3 = pltpu.async_copy(din_hbm.at[pl.ds(idx_s0, _IDXN)], ii, ls1)
    c4 = pltpu.async_copy(dout_hbm.at[pl.ds(idx_s0, _IDXN)], io, ls1)
    c1.wait(); c2.wait(); c3.wait(); c4.wait()

    accs = (acc0, acc1)
    lsems = (ls0, ls1)
    ssems = (ss0, ss1)

    def flat_chunk(t):
        return pl.ds((base_row + t * _K) * _D, _K * _D)

    pltpu.async_copy(x_hbm.at[flat_chunk(0)], acc0, ls0)

    def pair_body(tp, carry):
        for b in (0, 1):
            t = tp * 2 + b
            nb = 1 - b

            @pl.when(t + 1 < count)
            def _():
                @pl.when(t >= 1)
                def _():
                    pltpu.make_async_copy(
                        accs[nb], out_hbm.at[flat_chunk(0)], ssems[nb]).wait()

                pltpu.async_copy(x_hbm.at[flat_chunk(t + 1)], accs[nb],
                                 lsems[nb])

            @pl.when(t < count)
            def _():
                pltpu.make_async_copy(
                    x_hbm.at[flat_chunk(0)], accs[b], lsems[b]).wait()
                _compute(zin_v, zout_v, ii, io, accs[b], off + t * _K,
                         _K // _L)
                pltpu.async_copy(accs[b], out_hbm.at[flat_chunk(t)], ssems[b])

        return carry

    lax.fori_loop(0, 13, pair_body, 0, unroll=False)

    pltpu.make_async_copy(acc0, out_hbm.at[flat_chunk(0)], ss0).wait()
    pltpu.make_async_copy(acc1, out_hbm.at[flat_chunk(0)], ss1).wait()

    @pl.when(wid == _NW - 1)
    def _():
        tail = pl.ds(_FULL * _K * _D, _TAIL * _D)
        pltpu.sync_copy(x_hbm.at[tail], acc0.at[pl.ds(0, _TAIL * _D)])
        _compute(zin_v, zout_v, ii, io, acc0, off + 24 * _K, _TAIL // _L)
        pltpu.sync_copy(acc0.at[pl.ds(0, _TAIL * _D)], out_hbm.at[tail])


@jax.jit
def _centrality(x2, din, dout, z_in, z_out):
    mesh = plsc.VectorSubcoreMesh(core_axis_name="c", subcore_axis_name="s")
    fn = functools.partial(
        pl.kernel,
        mesh=mesh,
        compiler_params=pltpu.CompilerParams(needs_layout_passes=False),
        out_type=jax.ShapeDtypeStruct((_N * _D,), jnp.float32),
        scratch_types=[
            pltpu.VMEM((_V * _D,), jnp.float32),
            pltpu.VMEM((_V * _D,), jnp.float32),
            pltpu.VMEM((_IDXN,), jnp.int32),
            pltpu.VMEM((_IDXN,), jnp.int32),
            pltpu.VMEM((_K * _D,), jnp.float32),
            pltpu.VMEM((_K * _D,), jnp.float32),
            pltpu.SemaphoreType.DMA,
            pltpu.SemaphoreType.DMA,
            pltpu.SemaphoreType.DMA,
            pltpu.SemaphoreType.DMA,
        ],
    )(_sc_body)
    return fn(x2, din, dout, z_in, z_out)


def kernel(x, in_degree, out_degree, z_in, z_out):
    x2 = x.reshape(_N * _D)
    out2 = _centrality(x2, in_degree.astype(jnp.int32),
                       out_degree.astype(jnp.int32), z_in.reshape(_V * _D),
                       z_out.reshape(_V * _D))
    return out2.reshape(x.shape)


# speedup vs baseline: 8.8206x; 1.0037x over previous
import functools

import jax
import jax.numpy as jnp
from jax import lax
from jax.experimental import pallas as pl
from jax.experimental.pallas import tpu as pltpu
from jax.experimental.pallas import tpu_sc as plsc

_N = 100000
_D = 128
_V = 257
_K = 128
_FULL = _N // _K
_TAIL = _N - _FULL * _K
_NW = 32
_IDXN = 3200
_L = 16


def _compute(zin_v, zout_v, ii, io, acc_c, ib, ngroups):
    iotas = [lax.iota(jnp.int32, _L) + k * _L for k in range(_D // _L)]

    def group(g, carry):
        din = ii[pl.ds(ib + g * _L, _L)] * _D
        dout = io[pl.ds(ib + g * _L, _L)] * _D
        gbase = g * _L * _D

        @plsc.parallel_loop(0, _L, unroll=4)
        def node(j):
            jsp = lax.broadcast(j, (_L,))
            bi = jnp.take_along_axis(din, jsp, axis=0)
            bo = jnp.take_along_axis(dout, jsp, axis=0)
            for k in range(_D // _L):
                v = (plsc.load_gather(zin_v, [bi + iotas[k]])
                     + plsc.load_gather(zout_v, [bo + iotas[k]]))
                plsc.addupdate(acc_c.at[pl.ds(gbase + j * _D + k * _L, _L)], v)

        return carry

    lax.fori_loop(0, ngroups, group, 0, unroll=False)


def _sc_body(x_hbm, din_hbm, dout_hbm, zin_hbm, zout_hbm, out_hbm,
             zin_v, zout_v, ii, io, acc0, acc1, ls0, ls1, ss0, ss1):
    wid = lax.axis_index("s") * 2 + lax.axis_index("c")
    start_chunk = 24 * wid + jnp.minimum(wid, 13)
    count = jnp.where(wid < 13, 25, 24)
    base_row = start_chunk * _K
    idx_s0 = jnp.minimum(base_row, _N - _IDXN)
    off = base_row - idx_s0

    c1 = pltpu.async_copy(zin_hbm, zin_v, ls0)
    c2 = pltpu.async_copy(zout_hbm, zout_v, ls0)
    c3 = pltpu.async_copy(din_hbm.at[pl.ds(idx_s0, _IDXN)], ii, ls1)
    c4 = pltpu.async_copy(dout_hbm.at[pl.ds(idx_s0, _IDXN)], io, ls1)
    c1.wait(); c2.wait(); c3.wait(); c4.wait()

    accs = (acc0, acc1)
    lsems = (ls0, ls1)
    ssems = (ss0, ss1)

    def flat_chunk(t):
        return pl.ds((base_row + t * _K) * _D, _K * _D)

    pltpu.async_copy(x_hbm.at[flat_chunk(0)], acc0, ls0)

    def pair_body(tp, carry):
        for b in (0, 1):
            t = tp * 2 + b
            nb = 1 - b

            @pl.when(t + 1 < count)
            def _():
                @pl.when(t >= 1)
                def _():
                    pltpu.make_async_copy(
                        accs[nb], out_hbm.at[flat_chunk(0)], ssems[nb]).wait()

                pltpu.async_copy(x_hbm.at[flat_chunk(t + 1)], accs[nb],
                                 lsems[nb])

            @pl.when(t < count)
            def _():
                pltpu.make_async_copy(
                    x_hbm.at[flat_chunk(0)], accs[b], lsems[b]).wait()
                _compute(zin_v, zout_v, ii, io, accs[b], off + t * _K,
                         _K // _L)
                pltpu.async_copy(accs[b], out_hbm.at[flat_chunk(t)], ssems[b])

        return carry

    lax.fori_loop(0, 13, pair_body, 0, unroll=False)

    pltpu.make_async_copy(acc0, out_hbm.at[flat_chunk(0)], ss0).wait()
    pltpu.make_async_copy(acc1, out_hbm.at[flat_chunk(0)], ss1).wait()

    @pl.when(wid == _NW - 1)
    def _():
        tail = pl.ds(_FULL * _K * _D, _TAIL * _D)
        pltpu.sync_copy(x_hbm.at[tail], acc0.at[pl.ds(0, _TAIL * _D)])
        _compute(zin_v, zout_v, ii, io, acc0, off + 24 * _K, _TAIL // _L)
        pltpu.sync_copy(acc0.at[pl.ds(0, _TAIL * _D)], out_hbm.at[tail])


@jax.jit
def _centrality(x2, din, dout, z_in, z_out):
    mesh = plsc.VectorSubcoreMesh(core_axis_name="c", subcore_axis_name="s")
    fn = functools.partial(
        pl.kernel,
        mesh=mesh,
        compiler_params=pltpu.CompilerParams(needs_layout_passes=False),
        out_type=jax.ShapeDtypeStruct((_N * _D,), jnp.float32),
        scratch_types=[
            pltpu.VMEM((_V * _D,), jnp.float32),
            pltpu.VMEM((_V * _D,), jnp.float32),
            pltpu.VMEM((_IDXN,), jnp.int32),
            pltpu.VMEM((_IDXN,), jnp.int32),
            pltpu.VMEM((_K * _D,), jnp.float32),
            pltpu.VMEM((_K * _D,), jnp.float32),
            pltpu.SemaphoreType.DMA,
            pltpu.SemaphoreType.DMA,
            pltpu.SemaphoreType.DMA,
            pltpu.SemaphoreType.DMA,
        ],
    )(_sc_body)
    return fn(x2, din, dout, z_in, z_out)


def kernel(x, in_degree, out_degree, z_in, z_out):
    x2 = x.reshape(_N * _D)
    out2 = _centrality(x2, in_degree.astype(jnp.int32),
                       out_degree.astype(jnp.int32), z_in.reshape(_V * _D),
                       z_out.reshape(_V * _D))
    return out2.reshape(x.shape)


# speedup vs baseline: 10.3056x; 1.1684x over previous
import functools

import jax
import jax.numpy as jnp
from jax import lax
from jax.experimental import pallas as pl
from jax.experimental.pallas import tpu as pltpu
from jax.experimental.pallas import tpu_sc as plsc

_N = 100000
_D = 128
_V = 257
_K = 128
_FULL = _N // _K
_TAIL = _N - _FULL * _K
_NW = 32
_IDXN = 3200
_L = 16


def _compute(zin_v, zout_v, ii, io, acc_c, ib, ngroups):
    iotas = [lax.iota(jnp.int32, _L) + k * _L for k in range(_D // _L)]

    def group(g, carry):
        din = ii[pl.ds(ib + g * _L, _L)] * _D
        dout = io[pl.ds(ib + g * _L, _L)] * _D
        gbase = g * _L * _D

        @plsc.parallel_loop(0, _L, unroll=4)
        def node(j):
            jsp = lax.broadcast(j, (_L,))
            bi = jnp.take_along_axis(din, jsp, axis=0)
            bo = jnp.take_along_axis(dout, jsp, axis=0)
            for k in range(_D // _L):
                v = (plsc.load_gather(zin_v, [bi + iotas[k]])
                     + plsc.load_gather(zout_v, [bo + iotas[k]]))
                plsc.addupdate(acc_c.at[pl.ds(gbase + j * _D + k * _L, _L)], v)

        return carry

    lax.fori_loop(0, ngroups, group, 0, unroll=False)


def _sc_body(x_hbm, din_hbm, dout_hbm, zin_hbm, zout_hbm, out_hbm,
             zin_v, zout_v, ii, io, acc0, acc1, acc2,
             ls0, ls1, ls2, ss0, ss1, ss2):
    wid = lax.axis_index("s") * 2 + lax.axis_index("c")
    start_chunk = 24 * wid + jnp.minimum(wid, 13)
    count = jnp.where(wid < 13, 25, 24)
    base_row = start_chunk * _K
    idx_s0 = jnp.minimum(base_row, _N - _IDXN)
    off = base_row - idx_s0

    c1 = pltpu.async_copy(zin_hbm, zin_v, ls0)
    c2 = pltpu.async_copy(zout_hbm, zout_v, ls0)
    c3 = pltpu.async_copy(din_hbm.at[pl.ds(idx_s0, _IDXN)], ii, ls1)
    c4 = pltpu.async_copy(dout_hbm.at[pl.ds(idx_s0, _IDXN)], io, ls1)
    c1.wait(); c2.wait(); c3.wait(); c4.wait()

    accs = (acc0, acc1, acc2)
    lsems = (ls0, ls1, ls2)
    ssems = (ss0, ss1, ss2)

    def flat_chunk(t):
        return pl.ds((base_row + t * _K) * _D, _K * _D)

    pltpu.async_copy(x_hbm.at[flat_chunk(0)], acc0, ls0)

    def triple_body(tp, carry):
        for b in (0, 1, 2):
            t = tp * 3 + b
            nb = (b + 1) % 3

            @pl.when(t + 1 < count)
            def _():
                @pl.when(t >= 2)
                def _():
                    pltpu.make_async_copy(
                        accs[nb], out_hbm.at[flat_chunk(0)], ssems[nb]).wait()

                pltpu.async_copy(x_hbm.at[flat_chunk(t + 1)], accs[nb],
                                 lsems[nb])

            @pl.when(t < count)
            def _():
                pltpu.make_async_copy(
                    x_hbm.at[flat_chunk(0)], accs[b], lsems[b]).wait()
                _compute(zin_v, zout_v, ii, io, accs[b], off + t * _K,
                         _K // _L)
                pltpu.async_copy(accs[b], out_hbm.at[flat_chunk(t)], ssems[b])

        return carry

    lax.fori_loop(0, 9, triple_body, 0, unroll=False)

    pltpu.make_async_copy(acc0, out_hbm.at[flat_chunk(0)], ss0).wait()
    pltpu.make_async_copy(acc1, out_hbm.at[flat_chunk(0)], ss1).wait()
    pltpu.make_async_copy(acc2, out_hbm.at[flat_chunk(0)], ss2).wait()

    @pl.when(wid == _NW - 1)
    def _():
        tail = pl.ds(_FULL * _K * _D, _TAIL * _D)
        pltpu.sync_copy(x_hbm.at[tail], acc0.at[pl.ds(0, _TAIL * _D)])
        _compute(zin_v, zout_v, ii, io, acc0, off + 24 * _K, _TAIL // _L)
        pltpu.sync_copy(acc0.at[pl.ds(0, _TAIL * _D)], out_hbm.at[tail])


@jax.jit
def _centrality(x2, din, dout, z_in, z_out):
    mesh = plsc.VectorSubcoreMesh(core_axis_name="c", subcore_axis_name="s")
    fn = functools.partial(
        pl.kernel,
        mesh=mesh,
        compiler_params=pltpu.CompilerParams(needs_layout_passes=False),
        out_type=jax.ShapeDtypeStruct((_N * _D,), jnp.float32),
        scratch_types=[
            pltpu.VMEM((_V * _D,), jnp.float32),
            pltpu.VMEM((_V * _D,), jnp.float32),
            pltpu.VMEM((_IDXN,), jnp.int32),
            pltpu.VMEM((_IDXN,), jnp.int32),
            pltpu.VMEM((_K * _D,), jnp.float32),
            pltpu.VMEM((_K * _D,), jnp.float32),
            pltpu.VMEM((_K * _D,), jnp.float32),
            pltpu.SemaphoreType.DMA,
            pltpu.SemaphoreType.DMA,
            pltpu.SemaphoreType.DMA,
            pltpu.SemaphoreType.DMA,
            pltpu.SemaphoreType.DMA,
            pltpu.SemaphoreType.DMA,
        ],
    )(_sc_body)
    return fn(x2, din, dout, z_in, z_out)


def kernel(x, in_degree, out_degree, z_in, z_out):
    x2 = x.reshape(_N * _D)
    out2 = _centrality(x2, in_degree.astype(jnp.int32),
                       out_degree.astype(jnp.int32), z_in.reshape(_V * _D),
                       z_out.reshape(_V * _D))
    return out2.reshape(x.shape)
